# baseline (speedup 1.0000x reference)
import functools

import numpy as np
import jax
import jax.numpy as jnp
from jax import lax
from jax.experimental import pallas as pl
from jax.experimental.pallas import tpu as pltpu
from jax.experimental.pallas import tpu_sc as plsc

B, S, V, D, P, T = 4, 4096, 100000, 768, 4096, 2
LN_EPS = 1e-12

NC, NS, L = 2, 16, 16
NW = NC * NS
N = B * S
TPW = N // NW
C = 64
NCHUNK = TPW // C
DV = D // L

_RSQRT_MAGIC = np.int32(0x5F3759DF)


def _xlane_sum(x, lanes):
    for k in (1, 2, 4, 8):
        perm = lax.bitwise_xor(lanes, np.int32(k))
        x = x + x.at[perm].get(mode="promise_in_bounds")
    return x


def _rsqrt_vec(v):
    i = lax.bitcast_convert_type(v, jnp.int32)
    i = _RSQRT_MAGIC - lax.shift_right_arithmetic(i, 1)
    y = lax.bitcast_convert_type(i, jnp.float32)
    half = np.float32(0.5) * v
    for _ in range(3):
        y = y * (np.float32(1.5) - half * y * y)
    return y


def _sc_body(ids_hbm, seg_hbm, tok_hbm, segtab_hbm, pos_hbm, gam_hbm, bet_hbm,
             out_hbm,
             idx_v, segv, segtab_v, gam_v, bet_v, rows_v, pos_v):
    wid = lax.axis_index("s") * NC + lax.axis_index("c")
    base = wid * TPW
    pos_base = (wid % (S // TPW)) * TPW

    pltpu.sync_copy(ids_hbm.at[pl.ds(base, TPW)], idx_v)
    pltpu.sync_copy(seg_hbm.at[pl.ds(base, TPW)], segv)
    pltpu.sync_copy(segtab_hbm, segtab_v)
    pltpu.sync_copy(gam_hbm, gam_v)
    pltpu.sync_copy(bet_hbm, bet_v)

    inv_d = np.float32(1.0 / D)
    lanes = lax.iota(jnp.int32, L)

    def chunk_body(c, _):
        off = pl.multiple_of(c * C, C)
        pltpu.sync_copy(tok_hbm.at[idx_v.at[pl.ds(off, C)]], rows_v)
        pltpu.sync_copy(pos_hbm.at[pl.ds(pos_base + off, C)], pos_v)

        def t_body(t, _):
            tvec = jnp.broadcast_to(off + t, (L,)).astype(jnp.int32)
            sidv = plsc.load_gather(segv, [tvec])
            segbase = sidv * np.int32(D) + lanes
            s1 = jnp.zeros((L,), jnp.float32)
            s2 = jnp.zeros((L,), jnp.float32)
            for j in range(DV):
                sl = pl.ds(j * L, L)
                segj = plsc.load_gather(segtab_v, [segbase + np.int32(j * L)])
                x = rows_v[t, sl] + pos_v[t, sl] + segj
                rows_v[t, sl] = x
                s1 = s1 + x
                s2 = s2 + x * x
            mu = _xlane_sum(s1, lanes) * inv_d
            var = _xlane_sum(s2, lanes) * inv_d - mu * mu
            ri = _rsqrt_vec(var + np.float32(LN_EPS))
            for j in range(DV):
                sl = pl.ds(j * L, L)
                x = rows_v[t, sl]
                rows_v[t, sl] = (x - mu) * ri * gam_v[sl] + bet_v[sl]
            return 0

        lax.fori_loop(0, C, t_body, 0)

        pltpu.sync_copy(rows_v, out_hbm.at[pl.ds(base + off, C)])
        return 0

    lax.fori_loop(0, NCHUNK, chunk_body, 0)


@functools.partial(jax.jit, static_argnums=())
def _bert_embed_sc(ids, segs, token_table, segment_table, pos_table, gamma, beta):
    mesh = plsc.VectorSubcoreMesh(core_axis_name="c", subcore_axis_name="s")
    k = pl.kernel(
        _sc_body,
        mesh=mesh,
        compiler_params=pltpu.CompilerParams(needs_layout_passes=False),
        out_type=jax.ShapeDtypeStruct((N, D), jnp.float32),
        scratch_types=[
            pltpu.VMEM((TPW,), jnp.int32),
            pltpu.VMEM((TPW,), jnp.int32),
            pltpu.VMEM((T * D,), jnp.float32),
            pltpu.VMEM((D,), jnp.float32),
            pltpu.VMEM((D,), jnp.float32),
            pltpu.VMEM((C, D), jnp.float32),
            pltpu.VMEM((C, D), jnp.float32),
        ],
    )
    return k(ids, segs, token_table, segment_table, pos_table, gamma, beta)


def kernel(input_ids, segment_ids, token_table, segment_table, pos_table, ln_gamma, ln_beta):
    ids = input_ids.reshape(-1).astype(jnp.int32)
    segs = segment_ids.reshape(-1).astype(jnp.int32)
    out = _bert_embed_sc(ids, segs, token_table, segment_table.reshape(-1),
                         pos_table, ln_gamma, ln_beta)
    return out.reshape(B, S, D)


# speedup vs baseline: 1.7186x; 1.7186x over previous
import functools

import numpy as np
import jax
import jax.numpy as jnp
from jax import lax
from jax.experimental import pallas as pl
from jax.experimental.pallas import tpu as pltpu
from jax.experimental.pallas import tpu_sc as plsc

B, S, V, D, P, T = 4, 4096, 100000, 768, 4096, 2
LN_EPS = 1e-12

NC, NS, L = 2, 16, 16
NW = NC * NS
N = B * S
TPW = N // NW
C = 16
NCHUNK = TPW // C
NBUF = 4
DV = D // L
GJ = 8
NG = DV // GJ

_RSQRT_MAGIC = np.int32(0x5F3759DF)


def _xlane_sum(x, lanes):
    for k in (1, 2, 4, 8):
        perm = lax.bitwise_xor(lanes, np.int32(k))
        x = x + x.at[perm].get(mode="promise_in_bounds")
    return x


def _rsqrt_vec(v):
    i = lax.bitcast_convert_type(v, jnp.int32)
    i = _RSQRT_MAGIC - lax.shift_right_arithmetic(i, 1)
    y = lax.bitcast_convert_type(i, jnp.float32)
    half = np.float32(0.5) * v
    for _ in range(3):
        y = y * (np.float32(1.5) - half * y * y)
    return y


def _sc_body(ids_hbm, seg_hbm, tok_hbm, segtab_hbm, pos_hbm, gam_hbm, bet_hbm,
             out_hbm,
             idx_v, segv, segtab_v, gam_v, bet_v, rows_v, pos_v,
             s1_v, s2_v, mus_v, ris_v,
             si0, si1, si2, si3, so0, so1, so2, so3):
    sin = (si0, si1, si2, si3)
    sout = (so0, so1, so2, so3)
    wid = lax.axis_index("s") * NC + lax.axis_index("c")
    base = wid * TPW
    pos_base = (wid % (S // TPW)) * TPW

    pltpu.sync_copy(ids_hbm.at[pl.ds(base, TPW)], idx_v)
    pltpu.sync_copy(seg_hbm.at[pl.ds(base, TPW)], segv)
    pltpu.sync_copy(segtab_hbm, segtab_v)
    pltpu.sync_copy(gam_hbm, gam_v)
    pltpu.sync_copy(bet_hbm, bet_v)

    lanes = lax.iota(jnp.int32, L)
    inv_d = np.float32(1.0 / D)

    def _off(c):
        if isinstance(c, int):
            return c * C
        return pl.multiple_of(c * C, C)

    def _in_descs(c, q):
        off = _off(c)
        return (
            pltpu.make_async_copy(tok_hbm.at[idx_v.at[pl.ds(off, C)]],
                                  rows_v.at[q], sin[q]),
            pltpu.make_async_copy(pos_hbm.at[pl.ds(pos_base + off, C)],
                                  pos_v.at[q], sin[q]),
        )

    def _issue_in(c, q):
        for dsc in _in_descs(c, q):
            dsc.start()

    def _wait_in(c, q):
        for dsc in _in_descs(c, q):
            dsc.wait()

    def _out_desc(c, q):
        off = _off(c)
        return pltpu.make_async_copy(rows_v.at[q],
                                     out_hbm.at[pl.ds(base + off, C)], sout[q])

    def _compute(c, q):
        off = _off(c)
        rv = rows_v.at[q]
        pv = pos_v.at[q]

        def a_body(t, _):
            tvec = jnp.broadcast_to(off + t, (L,)).astype(jnp.int32)
            sidv = plsc.load_gather(segv, [tvec])
            segbase = sidv * np.int32(D) + lanes
            s1a = jnp.zeros((L,), jnp.float32)
            s1b = jnp.zeros((L,), jnp.float32)
            s2a = jnp.zeros((L,), jnp.float32)
            s2b = jnp.zeros((L,), jnp.float32)
            for j in range(DV):
                sl = pl.ds(j * L, L)
                segj = plsc.load_gather(segtab_v, [segbase + np.int32(j * L)])
                x = rv[t, sl] + pv[t, sl] + segj
                rv[t, sl] = x
                if j % 2 == 0:
                    s1a = s1a + x
                    s2a = s2a + x * x
                else:
                    s1b = s1b + x
                    s2b = s2b + x * x
            s1_v[t, :] = s1a + s1b
            s2_v[t, :] = s2a + s2b
            return 0

        lax.fori_loop(0, C, a_body, 0)

        def b_body(i, _):
            for u in range(4):
                t = i * 4 + u
                tot1 = _xlane_sum(s1_v[t, :], lanes)
                tot2 = _xlane_sum(s2_v[t, :], lanes)
                mu = tot1 * inv_d
                var = tot2 * inv_d - mu * mu
                mus_v[t, :] = mu
                ris_v[t, :] = _rsqrt_vec(var + np.float32(LN_EPS))
            return 0

        lax.fori_loop(0, C // 4, b_body, 0)

        for jg in range(NG):
            gs = [gam_v[pl.ds((jg * GJ + u) * L, L)] for u in range(GJ)]
            bs = [bet_v[pl.ds((jg * GJ + u) * L, L)] for u in range(GJ)]

            def c_body(t, _):
                mu = mus_v[t, :]
                ri = ris_v[t, :]
                for u in range(GJ):
                    sl = pl.ds((jg * GJ + u) * L, L)
                    x = rv[t, sl]
                    rv[t, sl] = (x - mu) * ri * gs[u] + bs[u]
                return 0

            lax.fori_loop(0, C, c_body, 0)

    _issue_in(0, 0)
    _issue_in(1, 1)

    def g_body(g, _):
        for q in range(NBUF):
            c = g * NBUF + q
            _wait_in(c, q)
            qn = (q + 2) % NBUF
            if q < 2:
                @pl.when(g > 0)
                def _():
                    _out_desc(c - 2, qn).wait()

                _issue_in(c + 2, qn)
            else:
                _out_desc(c - 2, qn).wait()

                @pl.when(g < NCHUNK // NBUF - 1)
                def _():
                    _issue_in(c + 2, qn)

            _compute(c, q)
            _out_desc(c, q).start()
        return 0

    lax.fori_loop(0, NCHUNK // NBUF, g_body, 0)
    _out_desc(NCHUNK - 2, 2).wait()
    _out_desc(NCHUNK - 1, 3).wait()


@jax.jit
def _bert_embed_sc(ids, segs, token_table, segment_table, pos_table, gamma, beta):
    mesh = plsc.VectorSubcoreMesh(core_axis_name="c", subcore_axis_name="s")
    k = pl.kernel(
        _sc_body,
        mesh=mesh,
        compiler_params=pltpu.CompilerParams(needs_layout_passes=False),
        out_type=jax.ShapeDtypeStruct((N, D), jnp.float32),
        scratch_types=[
            pltpu.VMEM((TPW,), jnp.int32),
            pltpu.VMEM((TPW,), jnp.int32),
            pltpu.VMEM((T * D,), jnp.float32),
            pltpu.VMEM((D,), jnp.float32),
            pltpu.VMEM((D,), jnp.float32),
            pltpu.VMEM((NBUF, C, D), jnp.float32),
            pltpu.VMEM((NBUF, C, D), jnp.float32),
            pltpu.VMEM((C, L), jnp.float32),
            pltpu.VMEM((C, L), jnp.float32),
            pltpu.VMEM((C, L), jnp.float32),
            pltpu.VMEM((C, L), jnp.float32),
            pltpu.SemaphoreType.DMA,
            pltpu.SemaphoreType.DMA,
            pltpu.SemaphoreType.DMA,
            pltpu.SemaphoreType.DMA,
            pltpu.SemaphoreType.DMA,
            pltpu.SemaphoreType.DMA,
            pltpu.SemaphoreType.DMA,
            pltpu.SemaphoreType.DMA,
        ],
    )
    return k(ids, segs, token_table, segment_table, pos_table, gamma, beta)


def kernel(input_ids, segment_ids, token_table, segment_table, pos_table, ln_gamma, ln_beta):
    ids = input_ids.reshape(-1).astype(jnp.int32)
    segs = segment_ids.reshape(-1).astype(jnp.int32)
    out = _bert_embed_sc(ids, segs, token_table, segment_table.reshape(-1),
                         pos_table, ln_gamma, ln_beta)
    return out.reshape(B, S, D)


# speedup vs baseline: 1.7348x; 1.0094x over previous
import functools

import numpy as np
import jax
import jax.numpy as jnp
from jax import lax
from jax.experimental import pallas as pl
from jax.experimental.pallas import tpu as pltpu
from jax.experimental.pallas import tpu_sc as plsc

B, S, V, D, P, T = 4, 4096, 100000, 768, 4096, 2
LN_EPS = 1e-12

NC, NS, L = 2, 16, 16
NW = NC * NS
N = B * S
TPW = N // NW
C = 16
NCHUNK = TPW // C
NBUF = 4
NGRP = NCHUNK // NBUF
DV = D // L
GJ = 8
NG = DV // GJ

_RSQRT_MAGIC = np.int32(0x5F3759DF)


def _xlane_sum(x, lanes):
    for k in (1, 2, 4, 8):
        perm = lax.bitwise_xor(lanes, np.int32(k))
        x = x + x.at[perm].get(mode="promise_in_bounds")
    return x


def _rsqrt_vec(v):
    i = lax.bitcast_convert_type(v, jnp.int32)
    i = _RSQRT_MAGIC - lax.shift_right_arithmetic(i, 1)
    y = lax.bitcast_convert_type(i, jnp.float32)
    half = np.float32(0.5) * v
    for _ in range(2):
        y = y * (np.float32(1.5) - half * y * y)
    return y


def _sc_body(ids_hbm, seg_hbm, tok_hbm, segtab_hbm, pos_hbm, gam_hbm, bet_hbm,
             out_hbm,
             idx_v, segv, segtab_v, gam_v, bet_v, rows_v, pos_v,
             s1_v, s2_v, mus_v, ris_v,
             si0, si1, si2, si3, so0, so1, so2, so3):
    sin = (si0, si1, si2, si3)
    sout = (so0, so1, so2, so3)
    wid = lax.axis_index("s") * NC + lax.axis_index("c")
    base = wid * TPW
    pos_base = (wid % (S // TPW)) * TPW

    pltpu.sync_copy(ids_hbm.at[pl.ds(base, TPW)], idx_v)
    pltpu.sync_copy(seg_hbm.at[pl.ds(base, TPW)], segv)
    pltpu.sync_copy(segtab_hbm, segtab_v)
    pltpu.sync_copy(gam_hbm, gam_v)
    pltpu.sync_copy(bet_hbm, bet_v)

    lanes = lax.iota(jnp.int32, L)
    inv_d = np.float32(1.0 / D)

    def _off(c):
        if isinstance(c, int):
            return c * C
        return pl.multiple_of(c * C, C)

    def _in_descs(c, q):
        off = _off(c)
        return (
            pltpu.make_async_copy(tok_hbm.at[idx_v.at[pl.ds(off, C)]],
                                  rows_v.at[q], sin[q]),
            pltpu.make_async_copy(pos_hbm.at[pl.ds(pos_base + off, C)],
                                  pos_v.at[q], sin[q]),
        )

    def _issue_in(c, q):
        for dsc in _in_descs(c, q):
            dsc.start()

    def _wait_in(c, q):
        for dsc in _in_descs(c, q):
            dsc.wait()

    def _out_desc(c, q):
        off = _off(c)
        return pltpu.make_async_copy(rows_v.at[q],
                                     out_hbm.at[pl.ds(base + off, C)], sout[q])

    def _compute(c, q):
        off = _off(c)
        rv = rows_v.at[q]
        pv = pos_v.at[q]

        def a_body(t, _):
            tvec = jnp.broadcast_to(off + t, (L,)).astype(jnp.int32)
            sidv = plsc.load_gather(segv, [tvec])
            segbase = sidv * np.int32(D) + lanes
            s1a = jnp.zeros((L,), jnp.float32)
            s1b = jnp.zeros((L,), jnp.float32)
            s2a = jnp.zeros((L,), jnp.float32)
            s2b = jnp.zeros((L,), jnp.float32)
            for j in range(DV):
                sl = pl.ds(j * L, L)
                segj = plsc.load_gather(segtab_v, [segbase + np.int32(j * L)])
                x = rv[t, sl] + pv[t, sl] + segj
                rv[t, sl] = x
                if j % 2 == 0:
                    s1a = s1a + x
                    s2a = s2a + x * x
                else:
                    s1b = s1b + x
                    s2b = s2b + x * x
            s1_v[t, :] = s1a + s1b
            s2_v[t, :] = s2a + s2b
            return 0

        lax.fori_loop(0, C, a_body, 0)

        def b_body(i, _):
            for u in range(4):
                t = i * 4 + u
                tot1 = _xlane_sum(s1_v[t, :], lanes)
                tot2 = _xlane_sum(s2_v[t, :], lanes)
                mu = tot1 * inv_d
                var = tot2 * inv_d - mu * mu
                mus_v[t, :] = mu
                ris_v[t, :] = _rsqrt_vec(var + np.float32(LN_EPS))
            return 0

        lax.fori_loop(0, C // 4, b_body, 0)

        for jg in range(NG):
            gs = [gam_v[pl.ds((jg * GJ + u) * L, L)] for u in range(GJ)]
            bs = [bet_v[pl.ds((jg * GJ + u) * L, L)] for u in range(GJ)]

            def c_body(t, _):
                mu = mus_v[t, :]
                ri = ris_v[t, :]
                for u in range(GJ):
                    sl = pl.ds((jg * GJ + u) * L, L)
                    x = rv[t, sl]
                    rv[t, sl] = (x - mu) * ri * gs[u] + bs[u]
                return 0

            lax.fori_loop(0, C, c_body, 0)

    _issue_in(0, 0)
    _issue_in(1, 1)

    def g_body(g, _):
        for q in range(NBUF):
            c = g * NBUF + q
            _wait_in(c, q)
            qn = (q + 2) % NBUF
            if q < 2:
                @pl.when(g > 0)
                def _():
                    _out_desc(c - 2, qn).wait()

                _issue_in(c + 2, qn)
            else:
                _out_desc(c - 2, qn).wait()

                @pl.when(g < NGRP - 1)
                def _():
                    _issue_in(c + 2, qn)

            _compute(c, q)
            _out_desc(c, q).start()
        return 0

    lax.fori_loop(0, NGRP, g_body, 0)
    _out_desc(NCHUNK - 2, 2).wait()
    _out_desc(NCHUNK - 1, 3).wait()


@jax.jit
def _bert_embed_sc(ids, segs, token_table, segment_table, pos_table, gamma, beta):
    mesh = plsc.VectorSubcoreMesh(core_axis_name="c", subcore_axis_name="s")
    k = pl.kernel(
        _sc_body,
        mesh=mesh,
        compiler_params=pltpu.CompilerParams(needs_layout_passes=False),
        out_type=jax.ShapeDtypeStruct((N, D), jnp.float32),
        scratch_types=[
            pltpu.VMEM((TPW,), jnp.int32),
            pltpu.VMEM((TPW,), jnp.int32),
            pltpu.VMEM((T * D,), jnp.float32),
            pltpu.VMEM((D,), jnp.float32),
            pltpu.VMEM((D,), jnp.float32),
            pltpu.VMEM((NBUF, C, D), jnp.float32),
            pltpu.VMEM((NBUF, C, D), jnp.float32),
            pltpu.VMEM((C, L), jnp.float32),
            pltpu.VMEM((C, L), jnp.float32),
            pltpu.VMEM((C, L), jnp.float32),
            pltpu.VMEM((C, L), jnp.float32),
            pltpu.SemaphoreType.DMA,
            pltpu.SemaphoreType.DMA,
            pltpu.SemaphoreType.DMA,
            pltpu.SemaphoreType.DMA,
            pltpu.SemaphoreType.DMA,
            pltpu.SemaphoreType.DMA,
            pltpu.SemaphoreType.DMA,
            pltpu.SemaphoreType.DMA,
        ],
    )
    return k(ids, segs, token_table, segment_table, pos_table, gamma, beta)


def kernel(input_ids, segment_ids, token_table, segment_table, pos_table, ln_gamma, ln_beta):
    ids = input_ids.reshape(-1).astype(jnp.int32)
    segs = segment_ids.reshape(-1).astype(jnp.int32)
    out = _bert_embed_sc(ids, segs, token_table, segment_table.reshape(-1),
                         pos_table, ln_gamma, ln_beta)
    return out.reshape(B, S, D)


# speedup vs baseline: 1.8296x; 1.0546x over previous
import functools

import numpy as np
import jax
import jax.numpy as jnp
from jax import lax
from jax.experimental import pallas as pl
from jax.experimental.pallas import tpu as pltpu
from jax.experimental.pallas import tpu_sc as plsc

B, S, V, D, P, T = 4, 4096, 100000, 768, 4096, 2
LN_EPS = 1e-12

NC, NS, L = 2, 16, 16
NW = NC * NS
N = B * S
TPW = N // NW
C = 16
NCHUNK = TPW // C
NBUF = 4
NGRP = NCHUNK // NBUF
DV = D // L
GJ = 16
NG = DV // GJ

_RSQRT_MAGIC = np.int32(0x5F3759DF)


def _xlane_sum(x, lanes):
    for k in (1, 2, 4, 8):
        perm = lax.bitwise_xor(lanes, np.int32(k))
        x = x + x.at[perm].get(mode="promise_in_bounds")
    return x


def _rsqrt_vec(v):
    i = lax.bitcast_convert_type(v, jnp.int32)
    i = _RSQRT_MAGIC - lax.shift_right_arithmetic(i, 1)
    y = lax.bitcast_convert_type(i, jnp.float32)
    half = np.float32(0.5) * v
    for _ in range(2):
        y = y * (np.float32(1.5) - half * y * y)
    return y


def _sc_body(ids_hbm, seg_hbm, tok_hbm, segtab_hbm, pos_hbm, gam_hbm, bet_hbm,
             out_hbm,
             idx_v, segv, segtab_v, gam_v, bet_v, rows_v, pos_v,
             s1_v, s2_v, mus_v, ris_v,
             si0, si1, si2, si3, so0, so1, so2, so3):
    sin = (si0, si1, si2, si3)
    sout = (so0, so1, so2, so3)
    wid = lax.axis_index("s") * NC + lax.axis_index("c")
    base = wid * TPW
    pos_base = (wid % (S // TPW)) * TPW

    pltpu.sync_copy(ids_hbm.at[pl.ds(base, TPW)], idx_v)
    pltpu.sync_copy(seg_hbm.at[pl.ds(base, TPW)], segv)
    pltpu.sync_copy(segtab_hbm, segtab_v)
    pltpu.sync_copy(gam_hbm, gam_v)
    pltpu.sync_copy(bet_hbm, bet_v)

    lanes = lax.iota(jnp.int32, L)
    inv_d = np.float32(1.0 / D)

    def _off(c):
        if isinstance(c, int):
            return c * C
        return pl.multiple_of(c * C, C)

    def _in_descs(c, q):
        off = _off(c)
        return (
            pltpu.make_async_copy(tok_hbm.at[idx_v.at[pl.ds(off, C)]],
                                  rows_v.at[q], sin[q]),
            pltpu.make_async_copy(pos_hbm.at[pl.ds(pos_base + off, C)],
                                  pos_v.at[q], sin[q]),
        )

    def _issue_in(c, q):
        for dsc in _in_descs(c, q):
            dsc.start()

    def _wait_in(c, q):
        for dsc in _in_descs(c, q):
            dsc.wait()

    def _out_desc(c, q):
        off = _off(c)
        return pltpu.make_async_copy(rows_v.at[q],
                                     out_hbm.at[pl.ds(base + off, C)], sout[q])

    def _compute(c, q):
        off = _off(c)
        rv = rows_v.at[q]
        pv = pos_v.at[q]

        sgroup = segv[pl.ds(off, C)]

        def a_body(t, _):
            tvec = jnp.broadcast_to(t, (L,)).astype(jnp.int32)
            sidv = sgroup.at[tvec].get(mode="promise_in_bounds")
            segbase = sidv * np.int32(D) + lanes
            s1a = jnp.zeros((L,), jnp.float32)
            s1b = jnp.zeros((L,), jnp.float32)
            s2a = jnp.zeros((L,), jnp.float32)
            s2b = jnp.zeros((L,), jnp.float32)
            for j in range(DV):
                sl = pl.ds(j * L, L)
                segj = plsc.load_gather(segtab_v, [segbase + np.int32(j * L)])
                x = rv[t, sl] + pv[t, sl] + segj
                rv[t, sl] = x
                if j % 2 == 0:
                    s1a = s1a + x
                    s2a = s2a + x * x
                else:
                    s1b = s1b + x
                    s2b = s2b + x * x
            s1_v[t, :] = s1a + s1b
            s2_v[t, :] = s2a + s2b
            return 0

        lax.fori_loop(0, C, a_body, 0)

        def b_body(i, _):
            for u in range(4):
                t = i * 4 + u
                tot1 = _xlane_sum(s1_v[t, :], lanes)
                tot2 = _xlane_sum(s2_v[t, :], lanes)
                mu = tot1 * inv_d
                var = tot2 * inv_d - mu * mu
                mus_v[t, :] = mu
                ris_v[t, :] = _rsqrt_vec(var + np.float32(LN_EPS))
            return 0

        lax.fori_loop(0, C // 4, b_body, 0)

        for jg in range(NG):
            gs = [gam_v[pl.ds((jg * GJ + u) * L, L)] for u in range(GJ)]
            bs = [bet_v[pl.ds((jg * GJ + u) * L, L)] for u in range(GJ)]

            def c_body(t, _):
                mu = mus_v[t, :]
                ri = ris_v[t, :]
                for u in range(GJ):
                    sl = pl.ds((jg * GJ + u) * L, L)
                    x = rv[t, sl]
                    rv[t, sl] = (x - mu) * ri * gs[u] + bs[u]
                return 0

            lax.fori_loop(0, C, c_body, 0)

    _issue_in(0, 0)
    _issue_in(1, 1)

    def g_body(g, _):
        for q in range(NBUF):
            c = g * NBUF + q
            _wait_in(c, q)
            qn = (q + 2) % NBUF
            if q < 2:
                @pl.when(g > 0)
                def _():
                    _out_desc(c - 2, qn).wait()

                _issue_in(c + 2, qn)
            else:
                _out_desc(c - 2, qn).wait()

                @pl.when(g < NGRP - 1)
                def _():
                    _issue_in(c + 2, qn)

            _compute(c, q)
            _out_desc(c, q).start()
        return 0

    lax.fori_loop(0, NGRP, g_body, 0)
    _out_desc(NCHUNK - 2, 2).wait()
    _out_desc(NCHUNK - 1, 3).wait()


@jax.jit
def _bert_embed_sc(ids, segs, token_table, segment_table, pos_table, gamma, beta):
    mesh = plsc.VectorSubcoreMesh(core_axis_name="c", subcore_axis_name="s")
    k = pl.kernel(
        _sc_body,
        mesh=mesh,
        compiler_params=pltpu.CompilerParams(needs_layout_passes=False),
        out_type=jax.ShapeDtypeStruct((N, D), jnp.float32),
        scratch_types=[
            pltpu.VMEM((TPW,), jnp.int32),
            pltpu.VMEM((TPW,), jnp.int32),
            pltpu.VMEM((T * D,), jnp.float32),
            pltpu.VMEM((D,), jnp.float32),
            pltpu.VMEM((D,), jnp.float32),
            pltpu.VMEM((NBUF, C, D), jnp.float32),
            pltpu.VMEM((NBUF, C, D), jnp.float32),
            pltpu.VMEM((C, L), jnp.float32),
            pltpu.VMEM((C, L), jnp.float32),
            pltpu.VMEM((C, L), jnp.float32),
            pltpu.VMEM((C, L), jnp.float32),
            pltpu.SemaphoreType.DMA,
            pltpu.SemaphoreType.DMA,
            pltpu.SemaphoreType.DMA,
            pltpu.SemaphoreType.DMA,
            pltpu.SemaphoreType.DMA,
            pltpu.SemaphoreType.DMA,
            pltpu.SemaphoreType.DMA,
            pltpu.SemaphoreType.DMA,
        ],
    )
    return k(ids, segs, token_table, segment_table, pos_table, gamma, beta)


def kernel(input_ids, segment_ids, token_table, segment_table, pos_table, ln_gamma, ln_beta):
    ids = input_ids.reshape(-1).astype(jnp.int32)
    segs = segment_ids.reshape(-1).astype(jnp.int32)
    out = _bert_embed_sc(ids, segs, token_table, segment_table.reshape(-1),
                         pos_table, ln_gamma, ln_beta)
    return out.reshape(B, S, D)
